# Initial kernel scaffold; baseline (speedup 1.0000x reference)
import jax
import jax.numpy as jnp
from jax.experimental import pallas as pl


def kernel(coords, feats, box_class):
    raise NotImplementedError("write your pallas kernel here")


# speedup vs baseline: 10.3054x; 10.3054x over previous
import functools

import jax
import jax.numpy as jnp
from jax import lax
from jax.experimental import pallas as pl
from jax.experimental.pallas import tpu as pltpu
from jax.experimental.pallas import tpu_sc as plsc

RES_ = 128
C_ = 8
N_ = 500000
V_ = RES_ * RES_ * RES_
VQ_ = V_ // 4
NPAD_ = 512000
DUM_ = 8192
PTS_PER_TILE_ = NPAD_ // 32
PTS_PER_SC_TILE_ = NPAD_ // 16
ZCH_ = (VQ_ + DUM_) // 16
ZBUF_ = ZCH_ // 8


def _mesh():
    return plsc.VectorSubcoreMesh(
        core_axis_name="c", subcore_axis_name="s", num_cores=2, num_subcores=16
    )


def _idx_prep_body(xyz_hbm, idx_hbm, xv, yv, zv, i0, i1, i2, i3):
    w = lax.axis_index("c") * 16 + lax.axis_index("s")
    base = w * PTS_PER_TILE_
    pltpu.sync_copy(xyz_hbm.at[pl.ds(base, PTS_PER_TILE_)], xv)
    pltpu.sync_copy(xyz_hbm.at[pl.ds(NPAD_ + base, PTS_PER_TILE_)], yv)
    pltpu.sync_copy(xyz_hbm.at[pl.ds(2 * NPAD_ + base, PTS_PER_TILE_)], zv)
    lane = lax.iota(jnp.int32, 16)

    def body(i, _):
        x = xv[pl.ds(i * 16, 16)]
        y = yv[pl.ds(i * 16, 16)]
        z = zv[pl.ds(i * 16, 16)]
        xi = jnp.clip((x * float(RES_)).astype(jnp.int32), 0, RES_ - 1)
        yi = jnp.clip((y * float(RES_)).astype(jnp.int32), 0, RES_ - 1)
        zi = jnp.clip((z * float(RES_)).astype(jnp.int32), 0, RES_ - 1)
        flat = (xi * RES_ + yi) * RES_ + zi
        g = base + i * 16 + lane
        dum = VQ_ + (g & (DUM_ - 1))
        valid = g < N_
        for k, ref in enumerate((i0, i1, i2, i3)):
            d = flat - k * VQ_
            inr = lax.bitcast_convert_type(d, jnp.uint32) < jnp.uint32(VQ_)
            ikv = jnp.where(inr, d, dum)
            ref[pl.ds(i * 16, 16)] = jnp.where(valid, ikv, dum)
        return 0

    lax.fori_loop(0, PTS_PER_TILE_ // 16, body, 0)
    for k, ref in enumerate((i0, i1, i2, i3)):
        pltpu.sync_copy(ref, idx_hbm.at[pl.ds(k * NPAD_ + base, PTS_PER_TILE_)])


def _scatter_body(idx_hbm, vals_hbm, out_hbm, idx_v, val_v, zbuf, acc):
    c = lax.axis_index("c")
    s = lax.axis_index("s")

    def zb(i, _):
        zbuf[pl.ds(i * 16, 16)] = jnp.zeros((16,), jnp.float32)
        return 0

    lax.fori_loop(0, ZBUF_ // 16, zb, 0)
    for h in range(2):
        k = 2 * h + c
        pltpu.sync_copy(
            idx_hbm.at[pl.ds(k * NPAD_ + s * PTS_PER_SC_TILE_, PTS_PER_SC_TILE_)],
            idx_v,
        )
        for q in range(9):
            for r in range(8):
                pltpu.sync_copy(zbuf, acc.at[pl.ds(s * ZCH_ + r * ZBUF_, ZBUF_)])
            plsc.subcore_barrier()
            pltpu.sync_copy(
                vals_hbm.at[
                    pl.ds(q * NPAD_ + s * PTS_PER_SC_TILE_, PTS_PER_SC_TILE_)
                ],
                val_v,
            )
            pltpu.sync_copy(val_v, acc.at[idx_v], add=True)
            plsc.subcore_barrier()
            off = VQ_ // 16
            pltpu.sync_copy(
                acc.at[pl.ds(s * off, off)],
                out_hbm.at[pl.ds(q * V_ + k * VQ_ + s * off, off)],
            )
            plsc.subcore_barrier()


def _project_body(d_ref, p0_ref, p1_ref, p2_ref):
    i = pl.program_id(0)
    blk = d_ref[...]
    cnt = blk[8]
    mean = blk[0:8] / jnp.maximum(cnt, 1.0)[None]
    p1_ref[...] = jnp.max(mean, axis=2)
    p2_ref[...] = jnp.max(mean, axis=3)
    ph = jnp.max(mean, axis=1)

    @pl.when(i == 0)
    def _():
        p0_ref[...] = ph

    @pl.when(i != 0)
    def _():
        p0_ref[...] = jnp.maximum(p0_ref[...], ph)


def kernel(coords, feats, box_class):
    f32 = jnp.float32
    xyz = jnp.pad(coords[:, :3].astype(f32).T, ((0, 0), (0, NPAD_ - N_))).reshape(-1)
    vals = jnp.concatenate(
        [feats.astype(f32).T, jnp.ones((1, N_), f32)], axis=0
    )
    vals = jnp.pad(vals, ((0, 0), (0, NPAD_ - N_))).reshape(-1)

    idx2 = pl.kernel(
        _idx_prep_body,
        out_type=jax.ShapeDtypeStruct((4 * NPAD_,), jnp.int32),
        mesh=_mesh(),
        scratch_types=[
            pltpu.VMEM((PTS_PER_TILE_,), f32),
            pltpu.VMEM((PTS_PER_TILE_,), f32),
            pltpu.VMEM((PTS_PER_TILE_,), f32),
            pltpu.VMEM((PTS_PER_TILE_,), jnp.int32),
            pltpu.VMEM((PTS_PER_TILE_,), jnp.int32),
            pltpu.VMEM((PTS_PER_TILE_,), jnp.int32),
            pltpu.VMEM((PTS_PER_TILE_,), jnp.int32),
        ],
    )(xyz)

    dense = pl.kernel(
        _scatter_body,
        out_type=jax.ShapeDtypeStruct((9 * V_,), f32),
        mesh=_mesh(),
        scratch_types=[
            pltpu.VMEM((PTS_PER_SC_TILE_,), jnp.int32),
            pltpu.VMEM((PTS_PER_SC_TILE_,), f32),
            pltpu.VMEM((ZBUF_,), f32),
            pltpu.VMEM_SHARED((VQ_ + DUM_,), f32),
        ],
    )(idx2, vals)

    d4 = dense.reshape(9, RES_, RES_, RES_)
    grid = 16
    hs = RES_ // grid
    p0, p1, p2 = pl.pallas_call(
        _project_body,
        grid=(grid,),
        in_specs=[
            pl.BlockSpec((9, hs, RES_, RES_), lambda i: (0, i, 0, 0)),
        ],
        out_specs=[
            pl.BlockSpec((C_, RES_, RES_), lambda i: (0, 0, 0)),
            pl.BlockSpec((C_, hs, RES_), lambda i: (0, i, 0)),
            pl.BlockSpec((C_, hs, RES_), lambda i: (0, i, 0)),
        ],
        out_shape=[
            jax.ShapeDtypeStruct((C_, RES_, RES_), f32),
            jax.ShapeDtypeStruct((C_, RES_, RES_), f32),
            jax.ShapeDtypeStruct((C_, RES_, RES_), f32),
        ],
    )(d4)

    view_mask = jnp.stack([p0, p1, p2], axis=0)
    img_class = jnp.tile(box_class, 3)
    return view_mask, img_class
